# Initial kernel scaffold; baseline (speedup 1.0000x reference)
import jax
import jax.numpy as jnp
from jax.experimental import pallas as pl


def kernel(x, edge_index, W, a_src, a_dst, ln_g, ln_b, W1, b1, W2, b2):
    raise NotImplementedError("write your pallas kernel here")


# speedup vs baseline: 40.7990x; 40.7990x over previous
import functools

import jax
import jax.numpy as jnp
from jax import lax
from jax.experimental import pallas as pl
from jax.experimental.pallas import tpu as pltpu
from jax.experimental.pallas import tpu_sc as plsc

N = 10000
D = 128
H = 8
K = 16
FF = 512
E = 320000

NPAD = 10240
ROW = 144
DROW = 16
NC = 2
NS = 16
NW = NC * NS
EW = 10240
EPAD = NW * EW
CH = 128
NCH = EW // CH
RPT = NPAD // NS

BN = 512


def _tc_pre_body(x_ref, w2_ref, acat_ref, table_ref, dstt_ref, macc_ref):
    i = pl.program_id(0)
    z2 = jnp.dot(x_ref[...], w2_ref[...], preferred_element_type=jnp.float32)
    s = jnp.dot(z2, acat_ref[...], preferred_element_type=jnp.float32)
    table_ref[...] = jnp.concatenate([z2, s], axis=1)
    dstt_ref[...] = jnp.concatenate([s[:, 8:16], s[:, 0:8]], axis=1)

    part = jnp.broadcast_to(jnp.max(s, axis=0, keepdims=True), (8, 16))

    @pl.when(i == 0)
    def _():
        macc_ref[...] = jnp.full((8, 16), -jnp.inf, jnp.float32)

    macc_ref[...] = jnp.maximum(macc_ref[...], part)


def _tc_post_body(acc_ref, xp_ref, g_ref, b_ref, w1_ref, b1_ref, w2f_ref,
                  b2_ref, p_ref, out_ref):
    u = acc_ref[0] + acc_ref[1]
    unnorm = u[:, 0:128]
    den = u[:, 128:136]
    recip = 1.0 / (den + 1e-12)
    denf = jnp.dot(recip, p_ref[...], preferred_element_type=jnp.float32)
    gat = unnorm * denf
    hcol = jnp.where(gat > 0, gat, jnp.exp(gat) - 1.0)
    h = xp_ref[...] + hcol
    mu = jnp.mean(h, axis=1, keepdims=True)
    var = jnp.mean((h - mu) ** 2, axis=1, keepdims=True)
    ln = (h - mu) / jnp.sqrt(var + 1e-6) * g_ref[...] + b_ref[...]
    inter = jnp.maximum(
        jnp.dot(ln, w1_ref[...], preferred_element_type=jnp.float32)
        + b1_ref[...], 0.0)
    outp = jnp.dot(inter, w2f_ref[...], preferred_element_type=jnp.float32)
    out_ref[...] = outp + b2_ref[...] + h


def _sc_edge_body(table_h, dstt_h, src_h, dst_h, m_h, out_h,
                  sidx, didx, srow, drow, mv, wbuf, acc, sem1, sem2):
    c = lax.axis_index("c")
    s = lax.axis_index("s")
    wid = s * NC + c
    zero16 = jnp.zeros((16,), jnp.float32)

    def _zrow(i, _):
        for jj in range(ROW // 16):
            srow[i, pl.ds(jj * 16, 16)] = zero16
        return 0

    lax.fori_loop(0, CH, _zrow, 0)
    for j in range(RPT // CH):
        pltpu.sync_copy(srow, acc.at[pl.ds(s * RPT + j * CH, CH)])
    plsc.subcore_barrier()

    pltpu.sync_copy(m_h, mv)
    mval = mv[...]

    def _chunk(ch, _):
        base = wid * EW + ch * CH
        pltpu.sync_copy(src_h.at[pl.ds(base, CH)], sidx)
        pltpu.sync_copy(dst_h.at[pl.ds(base, CH)], didx)
        cp1 = pltpu.async_copy(table_h.at[sidx], srow, sem1)
        cp2 = pltpu.async_copy(dstt_h.at[didx], drow, sem2)
        cp1.wait()
        cp2.wait()

        def _edge(i, _):
            ss = srow[i, pl.ds(128, 16)]
            sd = drow[i, :]
            e = ss + sd
            e = jnp.where(e > 0, e, 0.01 * e)
            w = jnp.exp(e - mval)
            srow[i, pl.ds(128, 16)] = w
            for h in range(H):
                wh = w.at[jnp.full((16,), h, jnp.int32)].get(
                    mode="promise_in_bounds")
                srow[i, pl.ds(h * 16, 16)] = srow[i, pl.ds(h * 16, 16)] * wh
            return 0

        lax.fori_loop(0, CH, _edge, 0)
        pltpu.sync_copy(srow, acc.at[didx], add=True)
        return 0

    lax.fori_loop(0, NCH, _chunk, 0)
    plsc.subcore_barrier()
    pltpu.sync_copy(acc.at[pl.ds(s * RPT, RPT)],
                    out_h.at[c, pl.ds(s * RPT, RPT)])


def _make_sc_kernel():
    mesh = plsc.VectorSubcoreMesh(core_axis_name="c", subcore_axis_name="s")
    return functools.partial(
        pl.kernel, _sc_edge_body, mesh=mesh,
        out_type=jax.ShapeDtypeStruct((NC, NPAD, ROW), jnp.float32),
        scratch_types=[
            pltpu.VMEM((CH,), jnp.int32),
            pltpu.VMEM((CH,), jnp.int32),
            pltpu.VMEM((CH, ROW), jnp.float32),
            pltpu.VMEM((CH, DROW), jnp.float32),
            pltpu.VMEM((16,), jnp.float32),
            pltpu.VMEM((16,), jnp.float32),
            pltpu.VMEM_SHARED((NPAD, ROW), jnp.float32),
            pltpu.SemaphoreType.DMA,
            pltpu.SemaphoreType.DMA,
        ],
        compiler_params=pltpu.CompilerParams(use_tc_tiling_on_sc=False),
    )()


@jax.jit
def kernel(x, edge_index, W, a_src, a_dst, ln_g, ln_b, W1, b1, W2, b2):
    xp = jnp.pad(x, ((0, NPAD - N), (0, 0)))
    w2 = W.transpose(1, 0, 2).reshape(D, H * K)
    j = jnp.arange(D)
    h_of = j // K
    acat = (jnp.zeros((D, 16), jnp.float32)
            .at[j, h_of].set(a_src.reshape(-1))
            .at[j, h_of + 8].set(a_dst.reshape(-1)))
    pmat = jnp.zeros((H, D), jnp.float32).at[h_of, j].set(1.0)
    srcp = jnp.concatenate(
        [edge_index[0], jnp.full((EPAD - E,), N, jnp.int32)])
    dstp = jnp.concatenate(
        [edge_index[1], jnp.full((EPAD - E,), N, jnp.int32)])

    grid = NPAD // BN
    table, dstt, macc = pl.pallas_call(
        _tc_pre_body,
        grid=(grid,),
        in_specs=[
            pl.BlockSpec((BN, D), lambda i: (i, 0)),
            pl.BlockSpec((D, D), lambda i: (0, 0)),
            pl.BlockSpec((D, 16), lambda i: (0, 0)),
        ],
        out_specs=[
            pl.BlockSpec((BN, ROW), lambda i: (i, 0)),
            pl.BlockSpec((BN, DROW), lambda i: (i, 0)),
            pl.BlockSpec((8, 16), lambda i: (0, 0)),
        ],
        out_shape=[
            jax.ShapeDtypeStruct((NPAD, ROW), jnp.float32),
            jax.ShapeDtypeStruct((NPAD, DROW), jnp.float32),
            jax.ShapeDtypeStruct((8, 16), jnp.float32),
        ],
        compiler_params=pltpu.CompilerParams(
            dimension_semantics=("arbitrary",)),
    )(xp, w2, acat)

    mx = jnp.max(macc, axis=0)
    mb = mx[0:8] + mx[8:16]
    mb = jnp.where(mb > 0, mb, 0.01 * mb)
    m16 = jnp.concatenate([mb, jnp.zeros((8,), jnp.float32)])

    acc = _make_sc_kernel()(table, dstt, srcp, dstp, m16)

    out = pl.pallas_call(
        _tc_post_body,
        grid=(grid,),
        in_specs=[
            pl.BlockSpec((NC, BN, ROW), lambda i: (0, i, 0)),
            pl.BlockSpec((BN, D), lambda i: (i, 0)),
            pl.BlockSpec((1, D), lambda i: (0, 0)),
            pl.BlockSpec((1, D), lambda i: (0, 0)),
            pl.BlockSpec((D, FF), lambda i: (0, 0)),
            pl.BlockSpec((1, FF), lambda i: (0, 0)),
            pl.BlockSpec((FF, D), lambda i: (0, 0)),
            pl.BlockSpec((1, D), lambda i: (0, 0)),
            pl.BlockSpec((H, D), lambda i: (0, 0)),
        ],
        out_specs=pl.BlockSpec((BN, D), lambda i: (i, 0)),
        out_shape=jax.ShapeDtypeStruct((NPAD, D), jnp.float32),
    )(acc, xp, ln_g.reshape(1, D), ln_b.reshape(1, D), W1,
      b1.reshape(1, FF), W2, b2.reshape(1, D), pmat)

    return out[:N]


# speedup vs baseline: 56.5813x; 1.3868x over previous
import functools

import jax
import jax.numpy as jnp
from jax import lax
from jax.experimental import pallas as pl
from jax.experimental.pallas import tpu as pltpu
from jax.experimental.pallas import tpu_sc as plsc

N = 10000
D = 128
H = 8
K = 16
FF = 512
E = 320000

NPAD = 10240
ROW = 144
DROW = 16
NC = 2
NS = 16
NW = NC * NS
EW = 10240
EPAD = NW * EW
CH = 64
NCH = EW // CH
RPT = NPAD // NS

BN = 512


def _tc_pre_body(x_ref, w2_ref, acat_ref, table_ref, dstt_ref, macc_ref):
    i = pl.program_id(0)
    z2 = jnp.dot(x_ref[...], w2_ref[...], preferred_element_type=jnp.float32)
    s = jnp.dot(z2, acat_ref[...], preferred_element_type=jnp.float32)
    table_ref[...] = jnp.concatenate([z2, s], axis=1)
    dstt_ref[...] = jnp.concatenate([s[:, 8:16], s[:, 0:8]], axis=1)

    part = jnp.broadcast_to(jnp.max(s, axis=0, keepdims=True), (8, 16))

    @pl.when(i == 0)
    def _():
        macc_ref[...] = jnp.full((8, 16), -jnp.inf, jnp.float32)

    macc_ref[...] = jnp.maximum(macc_ref[...], part)


def _tc_post_body(acc_ref, xp_ref, g_ref, b_ref, w1_ref, b1_ref, w2f_ref,
                  b2_ref, p_ref, out_ref):
    u = acc_ref[0] + acc_ref[1]
    unnorm = u[:, 0:128]
    den = u[:, 128:136]
    recip = 1.0 / (den + 1e-12)
    denf = jnp.dot(recip, p_ref[...], preferred_element_type=jnp.float32)
    gat = unnorm * denf
    hcol = jnp.where(gat > 0, gat, jnp.exp(gat) - 1.0)
    h = xp_ref[...] + hcol
    mu = jnp.mean(h, axis=1, keepdims=True)
    var = jnp.mean((h - mu) ** 2, axis=1, keepdims=True)
    ln = (h - mu) / jnp.sqrt(var + 1e-6) * g_ref[...] + b_ref[...]
    inter = jnp.maximum(
        jnp.dot(ln, w1_ref[...], preferred_element_type=jnp.float32)
        + b1_ref[...], 0.0)
    outp = jnp.dot(inter, w2f_ref[...], preferred_element_type=jnp.float32)
    out_ref[...] = outp + b2_ref[...] + h


def _sc_edge_body(table_h, dstt_h, src_h, dst_h, m_h, out_h,
                  sidx_a, didx_a, sidx_b, didx_b,
                  srow_a, drow_a, srow_b, drow_b, mv, acc,
                  gs_a, gs_b, is_a, is_b):
    c = lax.axis_index("c")
    s = lax.axis_index("s")
    wid = s * NC + c
    zero16 = jnp.zeros((16,), jnp.float32)

    def _zrow(i, _):
        for jj in range(ROW // 16):
            srow_a[i, pl.ds(jj * 16, 16)] = zero16
        return 0

    lax.fori_loop(0, CH, _zrow, 0)
    for j in range(RPT // CH):
        pltpu.sync_copy(srow_a, acc.at[pl.ds(s * RPT + j * CH, CH)])
    plsc.subcore_barrier()

    pltpu.sync_copy(m_h, mv)
    mval = mv[...]

    def _issue_idx(ch, sidx, didx, sem):
        pltpu.async_copy(src_h.at[wid, ch], sidx, sem)
        pltpu.async_copy(dst_h.at[wid, ch], didx, sem)

    def _wait_idx(ch, sidx, didx, sem):
        pltpu.make_async_copy(src_h.at[wid, ch], sidx, sem).wait()
        pltpu.make_async_copy(dst_h.at[wid, ch], didx, sem).wait()

    def _issue_g(sidx, didx, srow, drow, sem):
        pltpu.async_copy(table_h.at[sidx], srow, sem)
        pltpu.async_copy(dstt_h.at[didx], drow, sem)

    def _wait_g(sidx, didx, srow, drow, sem):
        pltpu.make_async_copy(table_h.at[sidx], srow, sem).wait()
        pltpu.make_async_copy(dstt_h.at[didx], drow, sem).wait()

    def _compute(srow, drow):
        def _edge(i, _):
            ss = srow[i, pl.ds(128, 16)]
            sd = drow[i, :]
            e = ss + sd
            e = jnp.where(e > 0, e, 0.01 * e)
            w = jnp.exp(e - mval)
            srow[i, pl.ds(128, 16)] = w
            for h in range(H):
                wh = w.at[jnp.full((16,), h, jnp.int32)].get(
                    mode="promise_in_bounds")
                srow[i, pl.ds(h * 16, 16)] = srow[i, pl.ds(h * 16, 16)] * wh
            return 0

        lax.fori_loop(0, CH, _edge, 0)

    pltpu.sync_copy(src_h.at[wid, 0], sidx_a)
    pltpu.sync_copy(dst_h.at[wid, 0], didx_a)
    _issue_g(sidx_a, didx_a, srow_a, drow_a, gs_a)
    _issue_idx(1, sidx_b, didx_b, is_b)

    def _stage(ch, sidx, didx, srow, drow, gs, isem,
               sidx_n, didx_n, srow_n, drow_n, gs_n, isem_n):
        @pl.when(ch + 1 < NCH)
        def _():
            _wait_idx(ch + 1, sidx_n, didx_n, isem_n)
            _issue_g(sidx_n, didx_n, srow_n, drow_n, gs_n)

        _wait_g(sidx, didx, srow, drow, gs)
        _compute(srow, drow)
        pltpu.sync_copy(srow, acc.at[didx], add=True)

        @pl.when(ch + 2 < NCH)
        def _():
            _issue_idx(ch + 2, sidx, didx, isem)

    def _pair(t, _):
        ch0 = 2 * t
        _stage(ch0, sidx_a, didx_a, srow_a, drow_a, gs_a, is_a,
               sidx_b, didx_b, srow_b, drow_b, gs_b, is_b)
        _stage(ch0 + 1, sidx_b, didx_b, srow_b, drow_b, gs_b, is_b,
               sidx_a, didx_a, srow_a, drow_a, gs_a, is_a)
        return 0

    lax.fori_loop(0, NCH // 2, _pair, 0)
    plsc.subcore_barrier()
    pltpu.sync_copy(acc.at[pl.ds(s * RPT, RPT)],
                    out_h.at[c, pl.ds(s * RPT, RPT)])


def _make_sc_kernel():
    mesh = plsc.VectorSubcoreMesh(core_axis_name="c", subcore_axis_name="s")
    return functools.partial(
        pl.kernel, _sc_edge_body, mesh=mesh,
        out_type=jax.ShapeDtypeStruct((NC, NPAD, ROW), jnp.float32),
        scratch_types=[
            pltpu.VMEM((CH,), jnp.int32),
            pltpu.VMEM((CH,), jnp.int32),
            pltpu.VMEM((CH,), jnp.int32),
            pltpu.VMEM((CH,), jnp.int32),
            pltpu.VMEM((CH, ROW), jnp.float32),
            pltpu.VMEM((CH, DROW), jnp.float32),
            pltpu.VMEM((CH, ROW), jnp.float32),
            pltpu.VMEM((CH, DROW), jnp.float32),
            pltpu.VMEM((16,), jnp.float32),
            pltpu.VMEM_SHARED((NPAD, ROW), jnp.float32),
            pltpu.SemaphoreType.DMA,
            pltpu.SemaphoreType.DMA,
            pltpu.SemaphoreType.DMA,
            pltpu.SemaphoreType.DMA,
        ],
        compiler_params=pltpu.CompilerParams(use_tc_tiling_on_sc=False),
    )()


@jax.jit
def kernel(x, edge_index, W, a_src, a_dst, ln_g, ln_b, W1, b1, W2, b2):
    xp = jnp.pad(x, ((0, NPAD - N), (0, 0)))
    w2 = W.transpose(1, 0, 2).reshape(D, H * K)
    j = jnp.arange(D)
    h_of = j // K
    acat = (jnp.zeros((D, 16), jnp.float32)
            .at[j, h_of].set(a_src.reshape(-1))
            .at[j, h_of + 8].set(a_dst.reshape(-1)))
    pmat = jnp.zeros((H, D), jnp.float32).at[h_of, j].set(1.0)
    srcp = jnp.concatenate(
        [edge_index[0], jnp.full((EPAD - E,), N, jnp.int32)]
    ).reshape(NW, NCH, CH)
    dstp = jnp.concatenate(
        [edge_index[1], jnp.full((EPAD - E,), N, jnp.int32)]
    ).reshape(NW, NCH, CH)

    grid = NPAD // BN
    table, dstt, macc = pl.pallas_call(
        _tc_pre_body,
        grid=(grid,),
        in_specs=[
            pl.BlockSpec((BN, D), lambda i: (i, 0)),
            pl.BlockSpec((D, D), lambda i: (0, 0)),
            pl.BlockSpec((D, 16), lambda i: (0, 0)),
        ],
        out_specs=[
            pl.BlockSpec((BN, ROW), lambda i: (i, 0)),
            pl.BlockSpec((BN, DROW), lambda i: (i, 0)),
            pl.BlockSpec((8, 16), lambda i: (0, 0)),
        ],
        out_shape=[
            jax.ShapeDtypeStruct((NPAD, ROW), jnp.float32),
            jax.ShapeDtypeStruct((NPAD, DROW), jnp.float32),
            jax.ShapeDtypeStruct((8, 16), jnp.float32),
        ],
        compiler_params=pltpu.CompilerParams(
            dimension_semantics=("arbitrary",)),
    )(xp, w2, acat)

    mx = jnp.max(macc, axis=0)
    mb = mx[0:8] + mx[8:16]
    mb = jnp.where(mb > 0, mb, 0.01 * mb)
    m16 = jnp.concatenate([mb, jnp.zeros((8,), jnp.float32)])

    acc = _make_sc_kernel()(table, dstt, srcp, dstp, m16)

    out = pl.pallas_call(
        _tc_post_body,
        grid=(grid,),
        in_specs=[
            pl.BlockSpec((NC, BN, ROW), lambda i: (0, i, 0)),
            pl.BlockSpec((BN, D), lambda i: (i, 0)),
            pl.BlockSpec((1, D), lambda i: (0, 0)),
            pl.BlockSpec((1, D), lambda i: (0, 0)),
            pl.BlockSpec((D, FF), lambda i: (0, 0)),
            pl.BlockSpec((1, FF), lambda i: (0, 0)),
            pl.BlockSpec((FF, D), lambda i: (0, 0)),
            pl.BlockSpec((1, D), lambda i: (0, 0)),
            pl.BlockSpec((H, D), lambda i: (0, 0)),
        ],
        out_specs=pl.BlockSpec((BN, D), lambda i: (i, 0)),
        out_shape=jax.ShapeDtypeStruct((NPAD, D), jnp.float32),
    )(acc, xp, ln_g.reshape(1, D), ln_b.reshape(1, D), W1,
      b1.reshape(1, FF), W2, b2.reshape(1, D), pmat)

    return out[:N]


# speedup vs baseline: 64.4149x; 1.1384x over previous
import functools

import jax
import jax.numpy as jnp
from jax import lax
from jax.experimental import pallas as pl
from jax.experimental.pallas import tpu as pltpu
from jax.experimental.pallas import tpu_sc as plsc

N = 10000
D = 128
H = 8
K = 16
FF = 512
E = 320000

NPAD = 10240
ROW = 144
TROW = 80
DROW = 16
NC = 2
NS = 16
NW = NC * NS
EW = 10240
EPAD = NW * EW
CH = 64
NCH = EW // CH
RPT = NPAD // NS

BN = 512


def _tc_pre_body(x_ref, w2_ref, acat_ref, table_ref, dstt_ref, macc_ref):
    i = pl.program_id(0)
    z2 = jnp.dot(x_ref[...], w2_ref[...], preferred_element_type=jnp.float32)
    s = jnp.dot(z2, acat_ref[...], preferred_element_type=jnp.float32)
    fb = jax.lax.bitcast_convert_type(z2, jnp.int32)
    rnd = fb + jnp.int32(0x7FFF) + ((fb >> 16) & 1)
    bfb = (rnd >> 16) & jnp.int32(0xFFFF)
    zp = bfb[:, 0:64] | (bfb[:, 64:128] << 16)
    sp = jax.lax.bitcast_convert_type(s[:, 0:8], jnp.int32)
    table_ref[...] = jnp.concatenate(
        [zp, sp, jnp.zeros((BN, 8), jnp.int32)], axis=1)
    dstt_ref[...] = jnp.concatenate([s[:, 8:16], s[:, 0:8]], axis=1)

    part = jnp.broadcast_to(jnp.max(s, axis=0, keepdims=True), (8, 16))

    @pl.when(i == 0)
    def _():
        macc_ref[...] = jnp.full((8, 16), -jnp.inf, jnp.float32)

    macc_ref[...] = jnp.maximum(macc_ref[...], part)


def _tc_post_body(acc_ref, xp_ref, g_ref, b_ref, w1_ref, b1_ref, w2f_ref,
                  b2_ref, p_ref, m_ref, out_ref):
    u = acc_ref[0] + acc_ref[1]
    unnorm = u[:, 0:128]
    den = u[:, 128:136]
    recip = 1.0 / (den + 1e-12)
    denf = jnp.dot(recip, p_ref[...], preferred_element_type=jnp.float32)
    gat = jnp.dot(unnorm * denf, m_ref[...],
                  preferred_element_type=jnp.float32)
    hcol = jnp.where(gat > 0, gat, jnp.exp(gat) - 1.0)
    h = xp_ref[...] + hcol
    mu = jnp.mean(h, axis=1, keepdims=True)
    var = jnp.mean((h - mu) ** 2, axis=1, keepdims=True)
    ln = (h - mu) / jnp.sqrt(var + 1e-6) * g_ref[...] + b_ref[...]
    inter = jnp.maximum(
        jnp.dot(ln, w1_ref[...], preferred_element_type=jnp.float32)
        + b1_ref[...], 0.0)
    outp = jnp.dot(inter, w2f_ref[...], preferred_element_type=jnp.float32)
    out_ref[...] = outp + b2_ref[...] + h


def _sc_edge_body(table_h, dstt_h, src_h, dst_h, m_h, out_h,
                  sidx_a, didx_a, sidx_b, didx_b,
                  spk_a, drow_a, spk_b, drow_b, srow, mv, acc,
                  gs_a, gs_b, is_a, is_b):
    c = lax.axis_index("c")
    s = lax.axis_index("s")
    wid = s * NC + c
    zero16 = jnp.zeros((16,), jnp.float32)
    himask = jnp.full((16,), -65536, jnp.int32)

    def _zrow(i, _):
        for jj in range(ROW // 16):
            srow[i, pl.ds(jj * 16, 16)] = zero16
        return 0

    lax.fori_loop(0, CH, _zrow, 0)
    for j in range(RPT // CH):
        pltpu.sync_copy(srow, acc.at[pl.ds(s * RPT + j * CH, CH)])
    plsc.subcore_barrier()

    pltpu.sync_copy(m_h, mv)
    mval = mv[...]

    def _issue_idx(ch, sidx, didx, sem):
        pltpu.async_copy(src_h.at[wid, ch], sidx, sem)
        pltpu.async_copy(dst_h.at[wid, ch], didx, sem)

    def _wait_idx(ch, sidx, didx, sem):
        pltpu.make_async_copy(src_h.at[wid, ch], sidx, sem).wait()
        pltpu.make_async_copy(dst_h.at[wid, ch], didx, sem).wait()

    def _issue_g(sidx, didx, spk, drow, sem):
        pltpu.async_copy(table_h.at[sidx], spk, sem)
        pltpu.async_copy(dstt_h.at[didx], drow, sem)

    def _wait_g(sidx, didx, spk, drow, sem):
        pltpu.make_async_copy(table_h.at[sidx], spk, sem).wait()
        pltpu.make_async_copy(dstt_h.at[didx], drow, sem).wait()

    def _compute(spk, drow):
        def _edge(i, _):
            ss = plsc.bitcast(spk[i, pl.ds(64, 16)], jnp.float32)
            sd = drow[i, :]
            e = ss + sd
            e = jnp.where(e > 0, e, 0.01 * e)
            w = jnp.exp(e - mval)
            srow[i, pl.ds(128, 16)] = w
            for g in range(4):
                vi = spk[i, pl.ds(g * 16, 16)]
                lo = plsc.bitcast(vi << 16, jnp.float32)
                hi = plsc.bitcast(vi & himask, jnp.float32)
                wlo = w.at[jnp.full((16,), g, jnp.int32)].get(
                    mode="promise_in_bounds")
                whi = w.at[jnp.full((16,), 4 + g, jnp.int32)].get(
                    mode="promise_in_bounds")
                srow[i, pl.ds(g * 32, 16)] = lo * wlo
                srow[i, pl.ds(g * 32 + 16, 16)] = hi * whi
            return 0

        lax.fori_loop(0, CH, _edge, 0)

    pltpu.sync_copy(src_h.at[wid, 0], sidx_a)
    pltpu.sync_copy(dst_h.at[wid, 0], didx_a)
    _issue_g(sidx_a, didx_a, spk_a, drow_a, gs_a)
    _issue_idx(1, sidx_b, didx_b, is_b)

    def _stage(ch, sidx, didx, spk, drow, gs, isem,
               sidx_n, didx_n, spk_n, drow_n, gs_n, isem_n):
        @pl.when(ch + 1 < NCH)
        def _():
            _wait_idx(ch + 1, sidx_n, didx_n, isem_n)
            _issue_g(sidx_n, didx_n, spk_n, drow_n, gs_n)

        _wait_g(sidx, didx, spk, drow, gs)
        _compute(spk, drow)
        pltpu.sync_copy(srow, acc.at[didx], add=True)

        @pl.when(ch + 2 < NCH)
        def _():
            _issue_idx(ch + 2, sidx, didx, isem)

    def _pair(t, _):
        ch0 = 2 * t
        _stage(ch0, sidx_a, didx_a, spk_a, drow_a, gs_a, is_a,
               sidx_b, didx_b, spk_b, drow_b, gs_b, is_b)
        _stage(ch0 + 1, sidx_b, didx_b, spk_b, drow_b, gs_b, is_b,
               sidx_a, didx_a, spk_a, drow_a, gs_a, is_a)
        return 0

    lax.fori_loop(0, NCH // 2, _pair, 0)
    plsc.subcore_barrier()
    pltpu.sync_copy(acc.at[pl.ds(s * RPT, RPT)],
                    out_h.at[c, pl.ds(s * RPT, RPT)])


def _make_sc_kernel():
    mesh = plsc.VectorSubcoreMesh(core_axis_name="c", subcore_axis_name="s")
    return functools.partial(
        pl.kernel, _sc_edge_body, mesh=mesh,
        out_type=jax.ShapeDtypeStruct((NC, NPAD, ROW), jnp.float32),
        scratch_types=[
            pltpu.VMEM((CH,), jnp.int32),
            pltpu.VMEM((CH,), jnp.int32),
            pltpu.VMEM((CH,), jnp.int32),
            pltpu.VMEM((CH,), jnp.int32),
            pltpu.VMEM((CH, TROW), jnp.int32),
            pltpu.VMEM((CH, DROW), jnp.float32),
            pltpu.VMEM((CH, TROW), jnp.int32),
            pltpu.VMEM((CH, DROW), jnp.float32),
            pltpu.VMEM((CH, ROW), jnp.float32),
            pltpu.VMEM((16,), jnp.float32),
            pltpu.VMEM_SHARED((NPAD, ROW), jnp.float32),
            pltpu.SemaphoreType.DMA,
            pltpu.SemaphoreType.DMA,
            pltpu.SemaphoreType.DMA,
            pltpu.SemaphoreType.DMA,
        ],
        compiler_params=pltpu.CompilerParams(
            use_tc_tiling_on_sc=False, needs_layout_passes=False),
    )()


@jax.jit
def kernel(x, edge_index, W, a_src, a_dst, ln_g, ln_b, W1, b1, W2, b2):
    xp = jnp.pad(x, ((0, NPAD - N), (0, 0)))
    w2 = W.transpose(1, 0, 2).reshape(D, H * K)
    j = jnp.arange(D)
    h_of = j // K
    acat = (jnp.zeros((D, 16), jnp.float32)
            .at[j, h_of].set(a_src.reshape(-1))
            .at[j, h_of + 8].set(a_dst.reshape(-1)))
    gg = j // 32
    hi_half = (j % 32) >= 16
    rr = j % 16
    orig = jnp.where(hi_half, 64 + 16 * gg + rr, 16 * gg + rr)
    head_pi = jnp.where(hi_half, 4 + gg, gg)
    pmat = jnp.zeros((H, D), jnp.float32).at[head_pi, j].set(1.0)
    unperm = jnp.zeros((D, D), jnp.float32).at[j, orig].set(1.0)
    srcp = jnp.concatenate(
        [edge_index[0], jnp.full((EPAD - E,), N, jnp.int32)]
    ).reshape(NW, NCH, CH)
    dstp = jnp.concatenate(
        [edge_index[1], jnp.full((EPAD - E,), N, jnp.int32)]
    ).reshape(NW, NCH, CH)

    grid = NPAD // BN
    table, dstt, macc = pl.pallas_call(
        _tc_pre_body,
        grid=(grid,),
        in_specs=[
            pl.BlockSpec((BN, D), lambda i: (i, 0)),
            pl.BlockSpec((D, D), lambda i: (0, 0)),
            pl.BlockSpec((D, 16), lambda i: (0, 0)),
        ],
        out_specs=[
            pl.BlockSpec((BN, TROW), lambda i: (i, 0)),
            pl.BlockSpec((BN, DROW), lambda i: (i, 0)),
            pl.BlockSpec((8, 16), lambda i: (0, 0)),
        ],
        out_shape=[
            jax.ShapeDtypeStruct((NPAD, TROW), jnp.int32),
            jax.ShapeDtypeStruct((NPAD, DROW), jnp.float32),
            jax.ShapeDtypeStruct((8, 16), jnp.float32),
        ],
        compiler_params=pltpu.CompilerParams(
            dimension_semantics=("arbitrary",)),
    )(xp, w2, acat)

    mx = jnp.max(macc, axis=0)
    mb = mx[0:8] + mx[8:16]
    mb = jnp.where(mb > 0, mb, 0.01 * mb)
    m16 = jnp.concatenate([mb, jnp.zeros((8,), jnp.float32)])

    acc = _make_sc_kernel()(table, dstt, srcp, dstp, m16)

    out = pl.pallas_call(
        _tc_post_body,
        grid=(grid,),
        in_specs=[
            pl.BlockSpec((NC, BN, ROW), lambda i: (0, i, 0)),
            pl.BlockSpec((BN, D), lambda i: (i, 0)),
            pl.BlockSpec((1, D), lambda i: (0, 0)),
            pl.BlockSpec((1, D), lambda i: (0, 0)),
            pl.BlockSpec((D, FF), lambda i: (0, 0)),
            pl.BlockSpec((1, FF), lambda i: (0, 0)),
            pl.BlockSpec((FF, D), lambda i: (0, 0)),
            pl.BlockSpec((1, D), lambda i: (0, 0)),
            pl.BlockSpec((H, D), lambda i: (0, 0)),
            pl.BlockSpec((D, D), lambda i: (0, 0)),
        ],
        out_specs=pl.BlockSpec((BN, D), lambda i: (i, 0)),
        out_shape=jax.ShapeDtypeStruct((NPAD, D), jnp.float32),
    )(acc, xp, ln_g.reshape(1, D), ln_b.reshape(1, D), W1,
      b1.reshape(1, FF), W2, b2.reshape(1, D), pmat, unperm)

    return out[:N]


# speedup vs baseline: 69.6862x; 1.0818x over previous
import functools

import jax
import jax.numpy as jnp
from jax import lax
from jax.experimental import pallas as pl
from jax.experimental.pallas import tpu as pltpu
from jax.experimental.pallas import tpu_sc as plsc

N = 10000
D = 128
H = 8
K = 16
FF = 512
E = 320000

NPAD = 10240
ROW = 144
TROW = 80
DROW = 16
NC = 2
NS = 16
NW = NC * NS
EW = 10240
EPAD = NW * EW
CH = 64
NCH = EW // CH
RPT = NPAD // NS

BN = 512


def _tc_pre_body(x_ref, w2_ref, acat_ref, table_ref, dstt_ref, macc_ref):
    i = pl.program_id(0)
    z2 = jnp.dot(x_ref[...], w2_ref[...], preferred_element_type=jnp.float32)
    s = jnp.dot(z2, acat_ref[...], preferred_element_type=jnp.float32)
    fb = jax.lax.bitcast_convert_type(z2, jnp.int32)
    rnd = fb + jnp.int32(0x7FFF) + ((fb >> 16) & 1)
    bfb = (rnd >> 16) & jnp.int32(0xFFFF)
    zp = bfb[:, 0:64] | (bfb[:, 64:128] << 16)
    sp = jax.lax.bitcast_convert_type(s[:, 0:8], jnp.int32)
    table_ref[...] = jnp.concatenate(
        [zp, sp, jnp.zeros((BN, 8), jnp.int32)], axis=1)
    dstt_ref[...] = jnp.concatenate([s[:, 8:16], s[:, 0:8]], axis=1)

    part = jnp.broadcast_to(jnp.max(s, axis=0, keepdims=True), (8, 16))

    @pl.when(i == 0)
    def _():
        macc_ref[...] = jnp.full((8, 16), -jnp.inf, jnp.float32)

    macc_ref[...] = jnp.maximum(macc_ref[...], part)


def _tc_post_body(acc_ref, xp_ref, g_ref, b_ref, w1_ref, b1_ref, w2f_ref,
                  b2_ref, p_ref, m_ref, out_ref):
    u = acc_ref[0] + acc_ref[1]
    unnorm = u[:, 0:128]
    den = u[:, 128:136]
    recip = 1.0 / (den + 1e-12)
    denf = jnp.dot(recip, p_ref[...], preferred_element_type=jnp.float32)
    gat = jnp.dot(unnorm * denf, m_ref[...],
                  preferred_element_type=jnp.float32)
    hcol = jnp.where(gat > 0, gat, jnp.exp(gat) - 1.0)
    h = xp_ref[...] + hcol
    mu = jnp.mean(h, axis=1, keepdims=True)
    var = jnp.mean((h - mu) ** 2, axis=1, keepdims=True)
    ln = (h - mu) / jnp.sqrt(var + 1e-6) * g_ref[...] + b_ref[...]
    inter = jnp.maximum(
        jnp.dot(ln, w1_ref[...], preferred_element_type=jnp.float32)
        + b1_ref[...], 0.0)
    outp = jnp.dot(inter, w2f_ref[...], preferred_element_type=jnp.float32)
    out_ref[...] = outp + b2_ref[...] + h


def _sc_edge_body(table_h, dstt_h, src_h, dst_h, m_h, out_h,
                  sidx_a, didx_a, sidx_b, didx_b, dsc_a, dsc_b,
                  spk_a, drow_a, spk_b, drow_b, srow_a, srow_b, mv, acc,
                  gs_a, gs_b, is_a, is_b, ss_a, ss_b):
    c = lax.axis_index("c")
    s = lax.axis_index("s")
    wid = s * NC + c
    zero16 = jnp.zeros((16,), jnp.float32)
    himask = jnp.full((16,), -65536, jnp.int32)

    def _zrow(i, _):
        for jj in range(ROW // 16):
            srow_a[i, pl.ds(jj * 16, 16)] = zero16
        return 0

    lax.fori_loop(0, CH, _zrow, 0)
    for j in range(RPT // CH):
        pltpu.sync_copy(srow_a, acc.at[pl.ds(s * RPT + j * CH, CH)])
    plsc.subcore_barrier()

    pltpu.sync_copy(m_h, mv)
    mval = mv[...]

    def _issue_idx(ch, sidx, didx, sem):
        pltpu.async_copy(src_h.at[wid, ch], sidx, sem)
        pltpu.async_copy(dst_h.at[wid, ch], didx, sem)

    def _wait_idx(ch, sidx, didx, sem):
        pltpu.make_async_copy(src_h.at[wid, ch], sidx, sem).wait()
        pltpu.make_async_copy(dst_h.at[wid, ch], didx, sem).wait()

    def _issue_g(sidx, didx, spk, drow, sem):
        pltpu.async_copy(table_h.at[sidx], spk, sem)
        pltpu.async_copy(dstt_h.at[didx], drow, sem)

    def _wait_g(sidx, didx, spk, drow, sem):
        pltpu.make_async_copy(table_h.at[sidx], spk, sem).wait()
        pltpu.make_async_copy(dstt_h.at[didx], drow, sem).wait()

    def _compute(spk, drow, srow):
        def _edge(i, _):
            ss = plsc.bitcast(spk[i, pl.ds(64, 16)], jnp.float32)
            sd = drow[i, :]
            e = ss + sd
            e = jnp.where(e > 0, e, 0.01 * e)
            w = jnp.exp(e - mval)
            srow[i, pl.ds(128, 16)] = w
            for g in range(4):
                vi = spk[i, pl.ds(g * 16, 16)]
                lo = plsc.bitcast(vi << 16, jnp.float32)
                hi = plsc.bitcast(vi & himask, jnp.float32)
                wlo = w.at[jnp.full((16,), g, jnp.int32)].get(
                    mode="promise_in_bounds")
                whi = w.at[jnp.full((16,), 4 + g, jnp.int32)].get(
                    mode="promise_in_bounds")
                srow[i, pl.ds(g * 32, 16)] = lo * wlo
                srow[i, pl.ds(g * 32 + 16, 16)] = hi * whi
            return 0

        lax.fori_loop(0, CH, _edge, 0, unroll=2)

    pltpu.sync_copy(src_h.at[wid, 0], sidx_a)
    pltpu.sync_copy(dst_h.at[wid, 0], didx_a)
    _issue_g(sidx_a, didx_a, spk_a, drow_a, gs_a)
    _issue_idx(1, sidx_b, didx_b, is_b)

    def _copy_idx(src_ref, dst_ref):
        for jj in range(CH // 16):
            dst_ref[pl.ds(jj * 16, 16)] = src_ref[pl.ds(jj * 16, 16)]

    def _stage(ch, sidx, didx, dsc, spk, drow, srow, gs, isem, ssem,
               sidx_n, didx_n, spk_n, drow_n, gs_n, isem_n):
        @pl.when(ch + 1 < NCH)
        def _():
            _wait_idx(ch + 1, sidx_n, didx_n, isem_n)
            _issue_g(sidx_n, didx_n, spk_n, drow_n, gs_n)

        _wait_g(sidx, didx, spk, drow, gs)

        @pl.when(ch >= 2)
        def _():
            pltpu.make_async_copy(srow, acc.at[dsc], ssem).wait()

        _compute(spk, drow, srow)
        _copy_idx(didx, dsc)
        pltpu.async_copy(srow, acc.at[dsc], ssem, add=True)

        @pl.when(ch + 2 < NCH)
        def _():
            _issue_idx(ch + 2, sidx, didx, isem)

    def _pair(t, _):
        ch0 = 2 * t
        _stage(ch0, sidx_a, didx_a, dsc_a, spk_a, drow_a, srow_a,
               gs_a, is_a, ss_a,
               sidx_b, didx_b, spk_b, drow_b, gs_b, is_b)
        _stage(ch0 + 1, sidx_b, didx_b, dsc_b, spk_b, drow_b, srow_b,
               gs_b, is_b, ss_b,
               sidx_a, didx_a, spk_a, drow_a, gs_a, is_a)
        return 0

    lax.fori_loop(0, NCH // 2, _pair, 0)
    pltpu.make_async_copy(srow_a, acc.at[dsc_a], ss_a).wait()
    pltpu.make_async_copy(srow_b, acc.at[dsc_b], ss_b).wait()
    plsc.subcore_barrier()
    pltpu.sync_copy(acc.at[pl.ds(s * RPT, RPT)],
                    out_h.at[c, pl.ds(s * RPT, RPT)])


def _make_sc_kernel():
    mesh = plsc.VectorSubcoreMesh(core_axis_name="c", subcore_axis_name="s")
    return functools.partial(
        pl.kernel, _sc_edge_body, mesh=mesh,
        out_type=jax.ShapeDtypeStruct((NC, NPAD, ROW), jnp.float32),
        scratch_types=[
            pltpu.VMEM((CH,), jnp.int32),
            pltpu.VMEM((CH,), jnp.int32),
            pltpu.VMEM((CH,), jnp.int32),
            pltpu.VMEM((CH,), jnp.int32),
            pltpu.VMEM((CH,), jnp.int32),
            pltpu.VMEM((CH,), jnp.int32),
            pltpu.VMEM((CH, TROW), jnp.int32),
            pltpu.VMEM((CH, DROW), jnp.float32),
            pltpu.VMEM((CH, TROW), jnp.int32),
            pltpu.VMEM((CH, DROW), jnp.float32),
            pltpu.VMEM((CH, ROW), jnp.float32),
            pltpu.VMEM((CH, ROW), jnp.float32),
            pltpu.VMEM((16,), jnp.float32),
            pltpu.VMEM_SHARED((NPAD, ROW), jnp.float32),
            pltpu.SemaphoreType.DMA,
            pltpu.SemaphoreType.DMA,
            pltpu.SemaphoreType.DMA,
            pltpu.SemaphoreType.DMA,
            pltpu.SemaphoreType.DMA,
            pltpu.SemaphoreType.DMA,
        ],
        compiler_params=pltpu.CompilerParams(
            use_tc_tiling_on_sc=False, needs_layout_passes=False),
    )()


@jax.jit
def kernel(x, edge_index, W, a_src, a_dst, ln_g, ln_b, W1, b1, W2, b2):
    xp = jnp.pad(x, ((0, NPAD - N), (0, 0)))
    w2 = W.transpose(1, 0, 2).reshape(D, H * K)
    j = jnp.arange(D)
    h_of = j // K
    acat = (jnp.zeros((D, 16), jnp.float32)
            .at[j, h_of].set(a_src.reshape(-1))
            .at[j, h_of + 8].set(a_dst.reshape(-1)))
    gg = j // 32
    hi_half = (j % 32) >= 16
    rr = j % 16
    orig = jnp.where(hi_half, 64 + 16 * gg + rr, 16 * gg + rr)
    head_pi = jnp.where(hi_half, 4 + gg, gg)
    pmat = jnp.zeros((H, D), jnp.float32).at[head_pi, j].set(1.0)
    unperm = jnp.zeros((D, D), jnp.float32).at[j, orig].set(1.0)
    srcp = jnp.concatenate(
        [edge_index[0], jnp.full((EPAD - E,), N, jnp.int32)]
    ).reshape(NW, NCH, CH)
    dstp = jnp.concatenate(
        [edge_index[1], jnp.full((EPAD - E,), N, jnp.int32)]
    ).reshape(NW, NCH, CH)

    grid = NPAD // BN
    table, dstt, macc = pl.pallas_call(
        _tc_pre_body,
        grid=(grid,),
        in_specs=[
            pl.BlockSpec((BN, D), lambda i: (i, 0)),
            pl.BlockSpec((D, D), lambda i: (0, 0)),
            pl.BlockSpec((D, 16), lambda i: (0, 0)),
        ],
        out_specs=[
            pl.BlockSpec((BN, TROW), lambda i: (i, 0)),
            pl.BlockSpec((BN, DROW), lambda i: (i, 0)),
            pl.BlockSpec((8, 16), lambda i: (0, 0)),
        ],
        out_shape=[
            jax.ShapeDtypeStruct((NPAD, TROW), jnp.int32),
            jax.ShapeDtypeStruct((NPAD, DROW), jnp.float32),
            jax.ShapeDtypeStruct((8, 16), jnp.float32),
        ],
        compiler_params=pltpu.CompilerParams(
            dimension_semantics=("arbitrary",)),
    )(xp, w2, acat)

    mx = jnp.max(macc, axis=0)
    mb = mx[0:8] + mx[8:16]
    mb = jnp.where(mb > 0, mb, 0.01 * mb)
    m16 = jnp.concatenate([mb, jnp.zeros((8,), jnp.float32)])

    acc = _make_sc_kernel()(table, dstt, srcp, dstp, m16)

    out = pl.pallas_call(
        _tc_post_body,
        grid=(grid,),
        in_specs=[
            pl.BlockSpec((NC, BN, ROW), lambda i: (0, i, 0)),
            pl.BlockSpec((BN, D), lambda i: (i, 0)),
            pl.BlockSpec((1, D), lambda i: (0, 0)),
            pl.BlockSpec((1, D), lambda i: (0, 0)),
            pl.BlockSpec((D, FF), lambda i: (0, 0)),
            pl.BlockSpec((1, FF), lambda i: (0, 0)),
            pl.BlockSpec((FF, D), lambda i: (0, 0)),
            pl.BlockSpec((1, D), lambda i: (0, 0)),
            pl.BlockSpec((H, D), lambda i: (0, 0)),
            pl.BlockSpec((D, D), lambda i: (0, 0)),
        ],
        out_specs=pl.BlockSpec((BN, D), lambda i: (i, 0)),
        out_shape=jax.ShapeDtypeStruct((NPAD, D), jnp.float32),
    )(acc, xp, ln_g.reshape(1, D), ln_b.reshape(1, D), W1,
      b1.reshape(1, FF), W2, b2.reshape(1, D), pmat, unperm)

    return out[:N]


# speedup vs baseline: 71.6735x; 1.0285x over previous
import functools

import jax
import jax.numpy as jnp
from jax import lax
from jax.experimental import pallas as pl
from jax.experimental.pallas import tpu as pltpu
from jax.experimental.pallas import tpu_sc as plsc

N = 10000
D = 128
H = 8
K = 16
FF = 512
E = 320000

NPAD = 10240
ROW = 144
TROW = 80
DROW = 16
NC = 2
NS = 16
NW = NC * NS
EW = 10240
EPAD = NW * EW
CH = 64
NCH = EW // CH
RPT = NPAD // NS

BN = 512


def _tc_pre_body(x_ref, w2_ref, acat_ref, table_ref, dstt_ref, macc_ref):
    i = pl.program_id(0)
    z2 = jnp.dot(x_ref[...], w2_ref[...], preferred_element_type=jnp.float32)
    s = jnp.dot(z2, acat_ref[...], preferred_element_type=jnp.float32)
    fb = jax.lax.bitcast_convert_type(z2, jnp.int32)
    rnd = fb + jnp.int32(0x7FFF) + ((fb >> 16) & 1)
    bfb = (rnd >> 16) & jnp.int32(0xFFFF)
    zp = bfb[:, 0:64] | (bfb[:, 64:128] << 16)
    sp = jax.lax.bitcast_convert_type(s[:, 0:8], jnp.int32)
    table_ref[...] = jnp.concatenate(
        [zp, sp, jnp.zeros((BN, 8), jnp.int32)], axis=1)
    dstt_ref[...] = jnp.concatenate([s[:, 8:16], s[:, 0:8]], axis=1)

    part = jnp.broadcast_to(jnp.max(s, axis=0, keepdims=True), (8, 16))

    @pl.when(i == 0)
    def _():
        macc_ref[...] = jnp.full((8, 16), -jnp.inf, jnp.float32)

    macc_ref[...] = jnp.maximum(macc_ref[...], part)


def _tc_post_body(acc_ref, xp_ref, g_ref, b_ref, w1_ref, b1_ref, w2f_ref,
                  b2_ref, p_ref, m_ref, out_ref):
    u = acc_ref[0] + acc_ref[1]
    unnorm = u[:, 0:128]
    den = u[:, 128:136]
    recip = 1.0 / (den + 1e-12)
    denf = jnp.dot(recip, p_ref[...], preferred_element_type=jnp.float32)
    gat = jnp.dot(unnorm * denf, m_ref[...],
                  preferred_element_type=jnp.float32)
    hcol = jnp.where(gat > 0, gat, jnp.exp(gat) - 1.0)
    h = xp_ref[...] + hcol
    mu = jnp.mean(h, axis=1, keepdims=True)
    var = jnp.mean((h - mu) ** 2, axis=1, keepdims=True)
    ln = (h - mu) / jnp.sqrt(var + 1e-6) * g_ref[...] + b_ref[...]
    inter = jnp.maximum(
        jnp.dot(ln, w1_ref[...], preferred_element_type=jnp.float32)
        + b1_ref[...], 0.0)
    outp = jnp.dot(inter, w2f_ref[...], preferred_element_type=jnp.float32)
    out_ref[...] = outp + b2_ref[...] + h


def _sc_edge_body(table_h, dstt_h, src_h, dst_h, m_h, out_h,
                  sidx_a, didx_a, sidx_b, didx_b, sidx_c, didx_c, dsc,
                  spk_a, drow_a, spk_b, drow_b, spk_c, drow_c, srow_a,
                  mv, acc, gs_a, gs_b, gs_c, is_a, is_b, is_c):
    c = lax.axis_index("c")
    s = lax.axis_index("s")
    wid = s * NC + c
    zero16 = jnp.zeros((16,), jnp.float32)
    himask = jnp.full((16,), -65536, jnp.int32)

    def _zrow(i, _):
        for jj in range(ROW // 16):
            srow_a[i, pl.ds(jj * 16, 16)] = zero16
        return 0

    lax.fori_loop(0, CH, _zrow, 0)
    for j in range(RPT // CH):
        pltpu.sync_copy(srow_a, acc.at[pl.ds(s * RPT + j * CH, CH)])
    plsc.subcore_barrier()

    pltpu.sync_copy(m_h, mv)
    mval = mv[...]

    def _issue_idx(ch, sidx, didx, sem):
        pltpu.async_copy(src_h.at[wid, ch], sidx, sem)
        pltpu.async_copy(dst_h.at[wid, ch], didx, sem)

    def _wait_idx(ch, sidx, didx, sem):
        pltpu.make_async_copy(src_h.at[wid, ch], sidx, sem).wait()
        pltpu.make_async_copy(dst_h.at[wid, ch], didx, sem).wait()

    def _issue_g(sidx, didx, spk, drow, sem):
        pltpu.async_copy(table_h.at[sidx], spk, sem)
        pltpu.async_copy(dstt_h.at[didx], drow, sem)

    def _wait_g(sidx, didx, spk, drow, sem):
        pltpu.make_async_copy(table_h.at[sidx], spk, sem).wait()
        pltpu.make_async_copy(dstt_h.at[didx], drow, sem).wait()

    def _compute(spk, drow, srow):
        def _edge(i, _):
            ss = plsc.bitcast(spk[i, pl.ds(64, 16)], jnp.float32)
            sd = drow[i, :]
            e = ss + sd
            e = jnp.where(e > 0, e, 0.01 * e)
            w = jnp.exp(e - mval)
            srow[i, pl.ds(128, 16)] = w
            for g in range(4):
                vi = spk[i, pl.ds(g * 16, 16)]
                lo = plsc.bitcast(vi << 16, jnp.float32)
                hi = plsc.bitcast(vi & himask, jnp.float32)
                wlo = w.at[jnp.full((16,), g, jnp.int32)].get(
                    mode="promise_in_bounds")
                whi = w.at[jnp.full((16,), 4 + g, jnp.int32)].get(
                    mode="promise_in_bounds")
                srow[i, pl.ds(g * 32, 16)] = lo * wlo
                srow[i, pl.ds(g * 32 + 16, 16)] = hi * whi
            return 0

        lax.fori_loop(0, CH, _edge, 0, unroll=2)

    pltpu.sync_copy(src_h.at[wid, 0], sidx_a)
    pltpu.sync_copy(dst_h.at[wid, 0], didx_a)
    _issue_g(sidx_a, didx_a, spk_a, drow_a, gs_a)
    pltpu.sync_copy(src_h.at[wid, 1], sidx_b)
    pltpu.sync_copy(dst_h.at[wid, 1], didx_b)
    _issue_g(sidx_b, didx_b, spk_b, drow_b, gs_b)
    _issue_idx(2, sidx_c, didx_c, is_c)

    def _stage(ch, sidx, didx, spk, drow, gs, isem,
               sidx_2, didx_2, spk_2, drow_2, gs_2, isem_2):
        @pl.when(ch + 2 < NCH)
        def _():
            _wait_idx(ch + 2, sidx_2, didx_2, isem_2)
            _issue_g(sidx_2, didx_2, spk_2, drow_2, gs_2)

        _wait_g(sidx, didx, spk, drow, gs)
        for jj in range(CH // 16):
            dsc[pl.ds(jj * 16, 16)] = didx[pl.ds(jj * 16, 16)]

        @pl.when(ch + 3 < NCH)
        def _():
            _issue_idx(ch + 3, sidx, didx, isem)

        _compute(spk, drow, srow_a)
        pltpu.sync_copy(srow_a, acc.at[dsc], add=True)

    def _triple(t, _):
        ch0 = 3 * t
        _stage(ch0, sidx_a, didx_a, spk_a, drow_a, gs_a, is_a,
               sidx_c, didx_c, spk_c, drow_c, gs_c, is_c)
        _stage(ch0 + 1, sidx_b, didx_b, spk_b, drow_b, gs_b, is_b,
               sidx_a, didx_a, spk_a, drow_a, gs_a, is_a)
        _stage(ch0 + 2, sidx_c, didx_c, spk_c, drow_c, gs_c, is_c,
               sidx_b, didx_b, spk_b, drow_b, gs_b, is_b)
        return 0

    lax.fori_loop(0, (NCH - 1) // 3, _triple, 0)
    _stage(NCH - 1, sidx_a, didx_a, spk_a, drow_a, gs_a, is_a,
           sidx_c, didx_c, spk_c, drow_c, gs_c, is_c)
    plsc.subcore_barrier()
    pltpu.sync_copy(acc.at[pl.ds(s * RPT, RPT)],
                    out_h.at[c, pl.ds(s * RPT, RPT)])


def _make_sc_kernel():
    mesh = plsc.VectorSubcoreMesh(core_axis_name="c", subcore_axis_name="s")
    return functools.partial(
        pl.kernel, _sc_edge_body, mesh=mesh,
        out_type=jax.ShapeDtypeStruct((NC, NPAD, ROW), jnp.float32),
        scratch_types=[
            pltpu.VMEM((CH,), jnp.int32),
            pltpu.VMEM((CH,), jnp.int32),
            pltpu.VMEM((CH,), jnp.int32),
            pltpu.VMEM((CH,), jnp.int32),
            pltpu.VMEM((CH,), jnp.int32),
            pltpu.VMEM((CH,), jnp.int32),
            pltpu.VMEM((CH,), jnp.int32),
            pltpu.VMEM((CH, TROW), jnp.int32),
            pltpu.VMEM((CH, DROW), jnp.float32),
            pltpu.VMEM((CH, TROW), jnp.int32),
            pltpu.VMEM((CH, DROW), jnp.float32),
            pltpu.VMEM((CH, TROW), jnp.int32),
            pltpu.VMEM((CH, DROW), jnp.float32),
            pltpu.VMEM((CH, ROW), jnp.float32),
            pltpu.VMEM((16,), jnp.float32),
            pltpu.VMEM_SHARED((NPAD, ROW), jnp.float32),
            pltpu.SemaphoreType.DMA,
            pltpu.SemaphoreType.DMA,
            pltpu.SemaphoreType.DMA,
            pltpu.SemaphoreType.DMA,
            pltpu.SemaphoreType.DMA,
            pltpu.SemaphoreType.DMA,
        ],
        compiler_params=pltpu.CompilerParams(
            use_tc_tiling_on_sc=False, needs_layout_passes=False),
    )()


@jax.jit
def kernel(x, edge_index, W, a_src, a_dst, ln_g, ln_b, W1, b1, W2, b2):
    xp = jnp.pad(x, ((0, NPAD - N), (0, 0)))
    w2 = W.transpose(1, 0, 2).reshape(D, H * K)
    j = jnp.arange(D)
    h_of = j // K
    acat = (jnp.zeros((D, 16), jnp.float32)
            .at[j, h_of].set(a_src.reshape(-1))
            .at[j, h_of + 8].set(a_dst.reshape(-1)))
    gg = j // 32
    hi_half = (j % 32) >= 16
    rr = j % 16
    orig = jnp.where(hi_half, 64 + 16 * gg + rr, 16 * gg + rr)
    head_pi = jnp.where(hi_half, 4 + gg, gg)
    pmat = jnp.zeros((H, D), jnp.float32).at[head_pi, j].set(1.0)
    unperm = jnp.zeros((D, D), jnp.float32).at[j, orig].set(1.0)
    srcp = jnp.concatenate(
        [edge_index[0], jnp.full((EPAD - E,), N, jnp.int32)]
    ).reshape(NW, NCH, CH)
    dstp = jnp.concatenate(
        [edge_index[1], jnp.full((EPAD - E,), N, jnp.int32)]
    ).reshape(NW, NCH, CH)

    grid = NPAD // BN
    table, dstt, macc = pl.pallas_call(
        _tc_pre_body,
        grid=(grid,),
        in_specs=[
            pl.BlockSpec((BN, D), lambda i: (i, 0)),
            pl.BlockSpec((D, D), lambda i: (0, 0)),
            pl.BlockSpec((D, 16), lambda i: (0, 0)),
        ],
        out_specs=[
            pl.BlockSpec((BN, TROW), lambda i: (i, 0)),
            pl.BlockSpec((BN, DROW), lambda i: (i, 0)),
            pl.BlockSpec((8, 16), lambda i: (0, 0)),
        ],
        out_shape=[
            jax.ShapeDtypeStruct((NPAD, TROW), jnp.int32),
            jax.ShapeDtypeStruct((NPAD, DROW), jnp.float32),
            jax.ShapeDtypeStruct((8, 16), jnp.float32),
        ],
        compiler_params=pltpu.CompilerParams(
            dimension_semantics=("arbitrary",)),
    )(xp, w2, acat)

    mx = jnp.max(macc, axis=0)
    mb = mx[0:8] + mx[8:16]
    mb = jnp.where(mb > 0, mb, 0.01 * mb)
    m16 = jnp.concatenate([mb, jnp.zeros((8,), jnp.float32)])

    acc = _make_sc_kernel()(table, dstt, srcp, dstp, m16)

    out = pl.pallas_call(
        _tc_post_body,
        grid=(grid,),
        in_specs=[
            pl.BlockSpec((NC, BN, ROW), lambda i: (0, i, 0)),
            pl.BlockSpec((BN, D), lambda i: (i, 0)),
            pl.BlockSpec((1, D), lambda i: (0, 0)),
            pl.BlockSpec((1, D), lambda i: (0, 0)),
            pl.BlockSpec((D, FF), lambda i: (0, 0)),
            pl.BlockSpec((1, FF), lambda i: (0, 0)),
            pl.BlockSpec((FF, D), lambda i: (0, 0)),
            pl.BlockSpec((1, D), lambda i: (0, 0)),
            pl.BlockSpec((H, D), lambda i: (0, 0)),
            pl.BlockSpec((D, D), lambda i: (0, 0)),
        ],
        out_specs=pl.BlockSpec((BN, D), lambda i: (i, 0)),
        out_shape=jax.ShapeDtypeStruct((NPAD, D), jnp.float32),
    )(acc, xp, ln_g.reshape(1, D), ln_b.reshape(1, D), W1,
      b1.reshape(1, FF), W2, b2.reshape(1, D), pmat, unperm)

    return out[:N]


# speedup vs baseline: 73.3439x; 1.0233x over previous
import functools

import jax
import jax.numpy as jnp
from jax import lax
from jax.experimental import pallas as pl
from jax.experimental.pallas import tpu as pltpu
from jax.experimental.pallas import tpu_sc as plsc

N = 10000
D = 128
H = 8
K = 16
FF = 512
E = 320000

NPAD = 10240
ROW = 144
TROW = 80
DROW = 16
NC = 2
NS = 16
NW = NC * NS
EW = 10240
EPAD = NW * EW
CH = 64
NCH = EW // CH
RPT = NPAD // NS

BN = 1024


def _tc_pre_body(x_ref, w2_ref, acat_ref, table_ref, dstt_ref, macc_ref):
    i = pl.program_id(0)
    z2 = jnp.dot(x_ref[...], w2_ref[...], preferred_element_type=jnp.float32)
    s = jnp.dot(z2, acat_ref[...], preferred_element_type=jnp.float32)
    fb = jax.lax.bitcast_convert_type(z2, jnp.int32)
    rnd = fb + jnp.int32(0x7FFF) + ((fb >> 16) & 1)
    bfb = (rnd >> 16) & jnp.int32(0xFFFF)
    zp = bfb[:, 0:64] | (bfb[:, 64:128] << 16)
    sp = jax.lax.bitcast_convert_type(s[:, 0:8], jnp.int32)
    table_ref[...] = jnp.concatenate(
        [zp, sp, jnp.zeros((BN, 8), jnp.int32)], axis=1)
    dstt_ref[...] = jnp.concatenate([s[:, 8:16], s[:, 0:8]], axis=1)

    part = jnp.broadcast_to(jnp.max(s, axis=0, keepdims=True), (8, 16))

    @pl.when(i == 0)
    def _():
        macc_ref[...] = jnp.full((8, 16), -jnp.inf, jnp.float32)

    macc_ref[...] = jnp.maximum(macc_ref[...], part)


def _tc_post_body(acc_ref, xp_ref, g_ref, b_ref, w1_ref, b1_ref, w2f_ref,
                  b2_ref, p_ref, m_ref, out_ref):
    u = acc_ref[0] + acc_ref[1]
    unnorm = u[:, 0:128]
    den = u[:, 128:136]
    recip = 1.0 / (den + 1e-12)
    denf = jnp.dot(recip, p_ref[...], preferred_element_type=jnp.float32)
    gat = jnp.dot(unnorm * denf, m_ref[...],
                  preferred_element_type=jnp.float32)
    hcol = jnp.where(gat > 0, gat, jnp.exp(gat) - 1.0)
    h = xp_ref[...] + hcol
    mu = jnp.mean(h, axis=1, keepdims=True)
    var = jnp.mean((h - mu) ** 2, axis=1, keepdims=True)
    ln = (h - mu) / jnp.sqrt(var + 1e-6) * g_ref[...] + b_ref[...]
    inter = jnp.maximum(
        jnp.dot(ln, w1_ref[...], preferred_element_type=jnp.float32)
        + b1_ref[...], 0.0)
    outp = jnp.dot(inter, w2f_ref[...], preferred_element_type=jnp.float32)
    out_ref[...] = outp + b2_ref[...] + h


def _sc_edge_body(table_h, dstt_h, src_h, dst_h, m_h, out_h,
                  sidx_a, didx_a, sidx_b, didx_b, sidx_c, didx_c, dsc,
                  spk_a, drow_a, spk_b, drow_b, spk_c, drow_c, srow_a,
                  mv, acc, gs_a, gs_b, gs_c, is_a, is_b, is_c):
    c = lax.axis_index("c")
    s = lax.axis_index("s")
    wid = s * NC + c
    zero16 = jnp.zeros((16,), jnp.float32)
    himask = jnp.full((16,), -65536, jnp.int32)

    def _zrow(i, _):
        for jj in range(ROW // 16):
            srow_a[i, pl.ds(jj * 16, 16)] = zero16
        return 0

    lax.fori_loop(0, CH, _zrow, 0)
    for j in range(RPT // CH):
        pltpu.sync_copy(srow_a, acc.at[pl.ds(s * RPT + j * CH, CH)])
    plsc.subcore_barrier()

    pltpu.sync_copy(m_h, mv)
    mval = mv[...]

    def _issue_idx(ch, sidx, didx, sem):
        pltpu.async_copy(src_h.at[wid, ch], sidx, sem)
        pltpu.async_copy(dst_h.at[wid, ch], didx, sem)

    def _wait_idx(ch, sidx, didx, sem):
        pltpu.make_async_copy(src_h.at[wid, ch], sidx, sem).wait()
        pltpu.make_async_copy(dst_h.at[wid, ch], didx, sem).wait()

    def _issue_g(sidx, didx, spk, drow, sem):
        pltpu.async_copy(table_h.at[sidx], spk, sem)
        pltpu.async_copy(dstt_h.at[didx], drow, sem)

    def _wait_g(sidx, didx, spk, drow, sem):
        pltpu.make_async_copy(table_h.at[sidx], spk, sem).wait()
        pltpu.make_async_copy(dstt_h.at[didx], drow, sem).wait()

    def _compute(spk, drow, srow):
        def _edge(i, _):
            ss = plsc.bitcast(spk[i, pl.ds(64, 16)], jnp.float32)
            sd = drow[i, :]
            e = ss + sd
            e = jnp.where(e > 0, e, 0.01 * e)
            w = jnp.exp(e - mval)
            srow[i, pl.ds(128, 16)] = w
            for g in range(4):
                vi = spk[i, pl.ds(g * 16, 16)]
                lo = plsc.bitcast(vi << 16, jnp.float32)
                hi = plsc.bitcast(vi & himask, jnp.float32)
                wlo = w.at[jnp.full((16,), g, jnp.int32)].get(
                    mode="promise_in_bounds")
                whi = w.at[jnp.full((16,), 4 + g, jnp.int32)].get(
                    mode="promise_in_bounds")
                srow[i, pl.ds(g * 32, 16)] = lo * wlo
                srow[i, pl.ds(g * 32 + 16, 16)] = hi * whi
            return 0

        lax.fori_loop(0, CH, _edge, 0, unroll=4)

    pltpu.sync_copy(src_h.at[wid, 0], sidx_a)
    pltpu.sync_copy(dst_h.at[wid, 0], didx_a)
    _issue_g(sidx_a, didx_a, spk_a, drow_a, gs_a)
    pltpu.sync_copy(src_h.at[wid, 1], sidx_b)
    pltpu.sync_copy(dst_h.at[wid, 1], didx_b)
    _issue_g(sidx_b, didx_b, spk_b, drow_b, gs_b)
    _issue_idx(2, sidx_c, didx_c, is_c)

    def _stage(ch, sidx, didx, spk, drow, gs, isem,
               sidx_2, didx_2, spk_2, drow_2, gs_2, isem_2):
        @pl.when(ch + 2 < NCH)
        def _():
            _wait_idx(ch + 2, sidx_2, didx_2, isem_2)
            _issue_g(sidx_2, didx_2, spk_2, drow_2, gs_2)

        _wait_g(sidx, didx, spk, drow, gs)
        for jj in range(CH // 16):
            dsc[pl.ds(jj * 16, 16)] = didx[pl.ds(jj * 16, 16)]

        @pl.when(ch + 3 < NCH)
        def _():
            _issue_idx(ch + 3, sidx, didx, isem)

        _compute(spk, drow, srow_a)
        pltpu.sync_copy(srow_a, acc.at[dsc], add=True)

    def _triple(t, _):
        ch0 = 3 * t
        _stage(ch0, sidx_a, didx_a, spk_a, drow_a, gs_a, is_a,
               sidx_c, didx_c, spk_c, drow_c, gs_c, is_c)
        _stage(ch0 + 1, sidx_b, didx_b, spk_b, drow_b, gs_b, is_b,
               sidx_a, didx_a, spk_a, drow_a, gs_a, is_a)
        _stage(ch0 + 2, sidx_c, didx_c, spk_c, drow_c, gs_c, is_c,
               sidx_b, didx_b, spk_b, drow_b, gs_b, is_b)
        return 0

    lax.fori_loop(0, (NCH - 1) // 3, _triple, 0)
    _stage(NCH - 1, sidx_a, didx_a, spk_a, drow_a, gs_a, is_a,
           sidx_c, didx_c, spk_c, drow_c, gs_c, is_c)
    plsc.subcore_barrier()
    pltpu.sync_copy(acc.at[pl.ds(s * RPT, RPT)],
                    out_h.at[c, pl.ds(s * RPT, RPT)])


def _make_sc_kernel():
    mesh = plsc.VectorSubcoreMesh(core_axis_name="c", subcore_axis_name="s")
    return functools.partial(
        pl.kernel, _sc_edge_body, mesh=mesh,
        out_type=jax.ShapeDtypeStruct((NC, NPAD, ROW), jnp.float32),
        scratch_types=[
            pltpu.VMEM((CH,), jnp.int32),
            pltpu.VMEM((CH,), jnp.int32),
            pltpu.VMEM((CH,), jnp.int32),
            pltpu.VMEM((CH,), jnp.int32),
            pltpu.VMEM((CH,), jnp.int32),
            pltpu.VMEM((CH,), jnp.int32),
            pltpu.VMEM((CH,), jnp.int32),
            pltpu.VMEM((CH, TROW), jnp.int32),
            pltpu.VMEM((CH, DROW), jnp.float32),
            pltpu.VMEM((CH, TROW), jnp.int32),
            pltpu.VMEM((CH, DROW), jnp.float32),
            pltpu.VMEM((CH, TROW), jnp.int32),
            pltpu.VMEM((CH, DROW), jnp.float32),
            pltpu.VMEM((CH, ROW), jnp.float32),
            pltpu.VMEM((16,), jnp.float32),
            pltpu.VMEM_SHARED((NPAD, ROW), jnp.float32),
            pltpu.SemaphoreType.DMA,
            pltpu.SemaphoreType.DMA,
            pltpu.SemaphoreType.DMA,
            pltpu.SemaphoreType.DMA,
            pltpu.SemaphoreType.DMA,
            pltpu.SemaphoreType.DMA,
        ],
        compiler_params=pltpu.CompilerParams(
            use_tc_tiling_on_sc=False, needs_layout_passes=False),
    )()


@jax.jit
def kernel(x, edge_index, W, a_src, a_dst, ln_g, ln_b, W1, b1, W2, b2):
    xp = jnp.pad(x, ((0, NPAD - N), (0, 0)))
    w2 = W.transpose(1, 0, 2).reshape(D, H * K)
    j = jnp.arange(D)
    h_of = j // K
    acat = (jnp.zeros((D, 16), jnp.float32)
            .at[j, h_of].set(a_src.reshape(-1))
            .at[j, h_of + 8].set(a_dst.reshape(-1)))
    gg = j // 32
    hi_half = (j % 32) >= 16
    rr = j % 16
    orig = jnp.where(hi_half, 64 + 16 * gg + rr, 16 * gg + rr)
    head_pi = jnp.where(hi_half, 4 + gg, gg)
    pmat = jnp.zeros((H, D), jnp.float32).at[head_pi, j].set(1.0)
    unperm = jnp.zeros((D, D), jnp.float32).at[j, orig].set(1.0)
    srcp = jnp.concatenate(
        [edge_index[0], jnp.full((EPAD - E,), N, jnp.int32)]
    ).reshape(NW, NCH, CH)
    dstp = jnp.concatenate(
        [edge_index[1], jnp.full((EPAD - E,), N, jnp.int32)]
    ).reshape(NW, NCH, CH)

    grid = NPAD // BN
    table, dstt, macc = pl.pallas_call(
        _tc_pre_body,
        grid=(grid,),
        in_specs=[
            pl.BlockSpec((BN, D), lambda i: (i, 0)),
            pl.BlockSpec((D, D), lambda i: (0, 0)),
            pl.BlockSpec((D, 16), lambda i: (0, 0)),
        ],
        out_specs=[
            pl.BlockSpec((BN, TROW), lambda i: (i, 0)),
            pl.BlockSpec((BN, DROW), lambda i: (i, 0)),
            pl.BlockSpec((8, 16), lambda i: (0, 0)),
        ],
        out_shape=[
            jax.ShapeDtypeStruct((NPAD, TROW), jnp.int32),
            jax.ShapeDtypeStruct((NPAD, DROW), jnp.float32),
            jax.ShapeDtypeStruct((8, 16), jnp.float32),
        ],
        compiler_params=pltpu.CompilerParams(
            dimension_semantics=("arbitrary",)),
    )(xp, w2, acat)

    mx = jnp.max(macc, axis=0)
    mb = mx[0:8] + mx[8:16]
    mb = jnp.where(mb > 0, mb, 0.01 * mb)
    m16 = jnp.concatenate([mb, jnp.zeros((8,), jnp.float32)])

    acc = _make_sc_kernel()(table, dstt, srcp, dstp, m16)

    out = pl.pallas_call(
        _tc_post_body,
        grid=(grid,),
        in_specs=[
            pl.BlockSpec((NC, BN, ROW), lambda i: (0, i, 0)),
            pl.BlockSpec((BN, D), lambda i: (i, 0)),
            pl.BlockSpec((1, D), lambda i: (0, 0)),
            pl.BlockSpec((1, D), lambda i: (0, 0)),
            pl.BlockSpec((D, FF), lambda i: (0, 0)),
            pl.BlockSpec((1, FF), lambda i: (0, 0)),
            pl.BlockSpec((FF, D), lambda i: (0, 0)),
            pl.BlockSpec((1, D), lambda i: (0, 0)),
            pl.BlockSpec((H, D), lambda i: (0, 0)),
            pl.BlockSpec((D, D), lambda i: (0, 0)),
        ],
        out_specs=pl.BlockSpec((BN, D), lambda i: (i, 0)),
        out_shape=jax.ShapeDtypeStruct((NPAD, D), jnp.float32),
    )(acc, xp, ln_g.reshape(1, D), ln_b.reshape(1, D), W1,
      b1.reshape(1, FF), W2, b2.reshape(1, D), pmat, unperm)

    return out[:N]


# speedup vs baseline: 73.5047x; 1.0022x over previous
import functools

import jax
import jax.numpy as jnp
from jax import lax
from jax.experimental import pallas as pl
from jax.experimental.pallas import tpu as pltpu
from jax.experimental.pallas import tpu_sc as plsc

N = 10000
D = 128
H = 8
K = 16
FF = 512
E = 320000

NPAD = 10240
ROW = 144
TROW = 80
DROW = 16
NC = 2
NS = 16
NW = NC * NS
EW = 10240
EPAD = NW * EW
CH = 64
NCH = EW // CH
RPT = NPAD // NS

BN = 1024


def _tc_pre_body(x_ref, w2_ref, acat_ref, table_ref, dstt_ref, macc_ref):
    i = pl.program_id(0)
    z2 = jnp.dot(x_ref[...], w2_ref[...], preferred_element_type=jnp.float32)
    s = jnp.dot(z2, acat_ref[...], preferred_element_type=jnp.float32)
    fb = jax.lax.bitcast_convert_type(z2, jnp.int32)
    rnd = fb + jnp.int32(0x7FFF) + ((fb >> 16) & 1)
    bfb = (rnd >> 16) & jnp.int32(0xFFFF)
    zp = bfb[:, 0:64] | (bfb[:, 64:128] << 16)
    sp = jax.lax.bitcast_convert_type(s[:, 0:8], jnp.int32)
    table_ref[...] = jnp.concatenate(
        [zp, sp, jnp.zeros((BN, 8), jnp.int32)], axis=1)
    dstt_ref[...] = jnp.concatenate([s[:, 8:16], s[:, 0:8]], axis=1)

    part = jnp.broadcast_to(jnp.max(s, axis=0, keepdims=True), (8, 16))

    @pl.when(i == 0)
    def _():
        macc_ref[...] = jnp.full((8, 16), -jnp.inf, jnp.float32)

    macc_ref[...] = jnp.maximum(macc_ref[...], part)


def _tc_post_body(acc_ref, xp_ref, g_ref, b_ref, w1_ref, b1_ref, w2f_ref,
                  b2_ref, p_ref, m_ref, out_ref):
    u = acc_ref[0] + acc_ref[1]
    unnorm = u[:, 0:128]
    den = u[:, 128:136]
    recip = 1.0 / (den + 1e-12)
    denf = jnp.dot(recip, p_ref[...], preferred_element_type=jnp.float32)
    gat = jnp.dot(unnorm * denf, m_ref[...],
                  preferred_element_type=jnp.float32)
    hcol = jnp.where(gat > 0, gat, jnp.exp(gat) - 1.0)
    h = xp_ref[...] + hcol
    mu = jnp.mean(h, axis=1, keepdims=True)
    var = jnp.mean((h - mu) ** 2, axis=1, keepdims=True)
    ln = (h - mu) / jnp.sqrt(var + 1e-6) * g_ref[...] + b_ref[...]
    inter = jnp.maximum(
        jnp.dot(ln, w1_ref[...], preferred_element_type=jnp.float32)
        + b1_ref[...], 0.0)
    outp = jnp.dot(inter, w2f_ref[...], preferred_element_type=jnp.float32)
    out_ref[...] = outp + b2_ref[...] + h


def _sc_edge_body(table_h, dstt_h, src_h, dst_h, m_h, out_h,
                  sidx_a, didx_a, sidx_b, didx_b, sidx_c, didx_c,
                  dsc_a, dsc_b, dsc_c,
                  spk_a, drow_a, spk_b, drow_b, spk_c, drow_c, srow_a,
                  mv, acc, gs_a, gs_b, gs_c, is_a, is_b, is_c, ssem):
    c = lax.axis_index("c")
    s = lax.axis_index("s")
    wid = s * NC + c
    zero16 = jnp.zeros((16,), jnp.float32)
    himask = jnp.full((16,), -65536, jnp.int32)

    def _zrow(i, _):
        for jj in range(ROW // 16):
            srow_a[i, pl.ds(jj * 16, 16)] = zero16
        return 0

    lax.fori_loop(0, CH, _zrow, 0)
    for j in range(RPT // CH):
        pltpu.sync_copy(srow_a, acc.at[pl.ds(s * RPT + j * CH, CH)])
    plsc.subcore_barrier()

    pltpu.sync_copy(m_h, mv)
    mval = mv[...]

    def _issue_idx(ch, sidx, didx, sem):
        pltpu.async_copy(src_h.at[wid, ch], sidx, sem)
        pltpu.async_copy(dst_h.at[wid, ch], didx, sem)

    def _wait_idx(ch, sidx, didx, sem):
        pltpu.make_async_copy(src_h.at[wid, ch], sidx, sem).wait()
        pltpu.make_async_copy(dst_h.at[wid, ch], didx, sem).wait()

    def _issue_g(sidx, didx, spk, drow, sem):
        pltpu.async_copy(table_h.at[sidx], spk, sem)
        pltpu.async_copy(dstt_h.at[didx], drow, sem)

    def _wait_g(sidx, didx, spk, drow, sem):
        pltpu.make_async_copy(table_h.at[sidx], spk, sem).wait()
        pltpu.make_async_copy(dstt_h.at[didx], drow, sem).wait()

    def _compute(spk, drow, srow):
        def _edge(i, _):
            ss = plsc.bitcast(spk[i, pl.ds(64, 16)], jnp.float32)
            sd = drow[i, :]
            e = ss + sd
            e = jnp.where(e > 0, e, 0.01 * e)
            w = jnp.exp(e - mval)
            srow[i, pl.ds(128, 16)] = w
            for g in range(4):
                vi = spk[i, pl.ds(g * 16, 16)]
                lo = plsc.bitcast(vi << 16, jnp.float32)
                hi = plsc.bitcast(vi & himask, jnp.float32)
                wlo = w.at[jnp.full((16,), g, jnp.int32)].get(
                    mode="promise_in_bounds")
                whi = w.at[jnp.full((16,), 4 + g, jnp.int32)].get(
                    mode="promise_in_bounds")
                srow[i, pl.ds(g * 32, 16)] = lo * wlo
                srow[i, pl.ds(g * 32 + 16, 16)] = hi * whi
            return 0

        lax.fori_loop(0, CH, _edge, 0, unroll=4)

    pltpu.sync_copy(src_h.at[wid, 0], sidx_a)
    pltpu.sync_copy(dst_h.at[wid, 0], didx_a)
    _issue_g(sidx_a, didx_a, spk_a, drow_a, gs_a)
    pltpu.sync_copy(src_h.at[wid, 1], sidx_b)
    pltpu.sync_copy(dst_h.at[wid, 1], didx_b)
    _issue_g(sidx_b, didx_b, spk_b, drow_b, gs_b)
    _issue_idx(2, sidx_c, didx_c, is_c)

    def _stage(ch, sidx, didx, dsc, spk, drow, gs, isem,
               sidx_2, didx_2, spk_2, drow_2, gs_2, isem_2, prev_dsc):
        @pl.when(ch + 2 < NCH)
        def _():
            _wait_idx(ch + 2, sidx_2, didx_2, isem_2)
            _issue_g(sidx_2, didx_2, spk_2, drow_2, gs_2)

        _wait_g(sidx, didx, spk, drow, gs)

        @pl.when(ch >= 1)
        def _():
            pltpu.make_async_copy(srow_a, acc.at[prev_dsc], ssem).wait()

        for jj in range(CH // 16):
            dsc[pl.ds(jj * 16, 16)] = didx[pl.ds(jj * 16, 16)]

        @pl.when(ch + 3 < NCH)
        def _():
            _issue_idx(ch + 3, sidx, didx, isem)

        _compute(spk, drow, srow_a)
        pltpu.async_copy(srow_a, acc.at[dsc], ssem, add=True)

    def _triple(t, _):
        ch0 = 3 * t
        _stage(ch0, sidx_a, didx_a, dsc_a, spk_a, drow_a, gs_a, is_a,
               sidx_c, didx_c, spk_c, drow_c, gs_c, is_c, dsc_c)
        _stage(ch0 + 1, sidx_b, didx_b, dsc_b, spk_b, drow_b, gs_b, is_b,
               sidx_a, didx_a, spk_a, drow_a, gs_a, is_a, dsc_a)
        _stage(ch0 + 2, sidx_c, didx_c, dsc_c, spk_c, drow_c, gs_c, is_c,
               sidx_b, didx_b, spk_b, drow_b, gs_b, is_b, dsc_b)
        return 0

    lax.fori_loop(0, (NCH - 1) // 3, _triple, 0)
    _stage(NCH - 1, sidx_a, didx_a, dsc_a, spk_a, drow_a, gs_a, is_a,
           sidx_c, didx_c, spk_c, drow_c, gs_c, is_c, dsc_c)
    pltpu.make_async_copy(srow_a, acc.at[dsc_a], ssem).wait()
    plsc.subcore_barrier()
    pltpu.sync_copy(acc.at[pl.ds(s * RPT, RPT)],
                    out_h.at[c, pl.ds(s * RPT, RPT)])


def _make_sc_kernel():
    mesh = plsc.VectorSubcoreMesh(core_axis_name="c", subcore_axis_name="s")
    return functools.partial(
        pl.kernel, _sc_edge_body, mesh=mesh,
        out_type=jax.ShapeDtypeStruct((NC, NPAD, ROW), jnp.float32),
        scratch_types=[
            pltpu.VMEM((CH,), jnp.int32),
            pltpu.VMEM((CH,), jnp.int32),
            pltpu.VMEM((CH,), jnp.int32),
            pltpu.VMEM((CH,), jnp.int32),
            pltpu.VMEM((CH,), jnp.int32),
            pltpu.VMEM((CH,), jnp.int32),
            pltpu.VMEM((CH,), jnp.int32),
            pltpu.VMEM((CH,), jnp.int32),
            pltpu.VMEM((CH,), jnp.int32),
            pltpu.VMEM((CH, TROW), jnp.int32),
            pltpu.VMEM((CH, DROW), jnp.float32),
            pltpu.VMEM((CH, TROW), jnp.int32),
            pltpu.VMEM((CH, DROW), jnp.float32),
            pltpu.VMEM((CH, TROW), jnp.int32),
            pltpu.VMEM((CH, DROW), jnp.float32),
            pltpu.VMEM((CH, ROW), jnp.float32),
            pltpu.VMEM((16,), jnp.float32),
            pltpu.VMEM_SHARED((NPAD, ROW), jnp.float32),
            pltpu.SemaphoreType.DMA,
            pltpu.SemaphoreType.DMA,
            pltpu.SemaphoreType.DMA,
            pltpu.SemaphoreType.DMA,
            pltpu.SemaphoreType.DMA,
            pltpu.SemaphoreType.DMA,
            pltpu.SemaphoreType.DMA,
        ],
        compiler_params=pltpu.CompilerParams(
            use_tc_tiling_on_sc=False, needs_layout_passes=False),
    )()


@jax.jit
def kernel(x, edge_index, W, a_src, a_dst, ln_g, ln_b, W1, b1, W2, b2):
    xp = jnp.pad(x, ((0, NPAD - N), (0, 0)))
    w2 = W.transpose(1, 0, 2).reshape(D, H * K)
    j = jnp.arange(D)
    h_of = j // K
    acat = (jnp.zeros((D, 16), jnp.float32)
            .at[j, h_of].set(a_src.reshape(-1))
            .at[j, h_of + 8].set(a_dst.reshape(-1)))
    gg = j // 32
    hi_half = (j % 32) >= 16
    rr = j % 16
    orig = jnp.where(hi_half, 64 + 16 * gg + rr, 16 * gg + rr)
    head_pi = jnp.where(hi_half, 4 + gg, gg)
    pmat = jnp.zeros((H, D), jnp.float32).at[head_pi, j].set(1.0)
    unperm = jnp.zeros((D, D), jnp.float32).at[j, orig].set(1.0)
    srcp = jnp.concatenate(
        [edge_index[0], jnp.full((EPAD - E,), N, jnp.int32)]
    ).reshape(NW, NCH, CH)
    dstp = jnp.concatenate(
        [edge_index[1], jnp.full((EPAD - E,), N, jnp.int32)]
    ).reshape(NW, NCH, CH)

    grid = NPAD // BN
    table, dstt, macc = pl.pallas_call(
        _tc_pre_body,
        grid=(grid,),
        in_specs=[
            pl.BlockSpec((BN, D), lambda i: (i, 0)),
            pl.BlockSpec((D, D), lambda i: (0, 0)),
            pl.BlockSpec((D, 16), lambda i: (0, 0)),
        ],
        out_specs=[
            pl.BlockSpec((BN, TROW), lambda i: (i, 0)),
            pl.BlockSpec((BN, DROW), lambda i: (i, 0)),
            pl.BlockSpec((8, 16), lambda i: (0, 0)),
        ],
        out_shape=[
            jax.ShapeDtypeStruct((NPAD, TROW), jnp.int32),
            jax.ShapeDtypeStruct((NPAD, DROW), jnp.float32),
            jax.ShapeDtypeStruct((8, 16), jnp.float32),
        ],
        compiler_params=pltpu.CompilerParams(
            dimension_semantics=("arbitrary",)),
    )(xp, w2, acat)

    mx = jnp.max(macc, axis=0)
    mb = mx[0:8] + mx[8:16]
    mb = jnp.where(mb > 0, mb, 0.01 * mb)
    m16 = jnp.concatenate([mb, jnp.zeros((8,), jnp.float32)])

    acc = _make_sc_kernel()(table, dstt, srcp, dstp, m16)

    out = pl.pallas_call(
        _tc_post_body,
        grid=(grid,),
        in_specs=[
            pl.BlockSpec((NC, BN, ROW), lambda i: (0, i, 0)),
            pl.BlockSpec((BN, D), lambda i: (i, 0)),
            pl.BlockSpec((1, D), lambda i: (0, 0)),
            pl.BlockSpec((1, D), lambda i: (0, 0)),
            pl.BlockSpec((D, FF), lambda i: (0, 0)),
            pl.BlockSpec((1, FF), lambda i: (0, 0)),
            pl.BlockSpec((FF, D), lambda i: (0, 0)),
            pl.BlockSpec((1, D), lambda i: (0, 0)),
            pl.BlockSpec((H, D), lambda i: (0, 0)),
            pl.BlockSpec((D, D), lambda i: (0, 0)),
        ],
        out_specs=pl.BlockSpec((BN, D), lambda i: (i, 0)),
        out_shape=jax.ShapeDtypeStruct((NPAD, D), jnp.float32),
    )(acc, xp, ln_g.reshape(1, D), ln_b.reshape(1, D), W1,
      b1.reshape(1, FF), W2, b2.reshape(1, D), pmat, unperm)

    return out[:N]
